# Initial kernel scaffold; baseline (speedup 1.0000x reference)
import jax
import jax.numpy as jnp
from jax.experimental import pallas as pl


def kernel(x, W0, b0, W1, b1, W2, b2, E, D0, db0, D1, db1, D2, db2):
    raise NotImplementedError("write your pallas kernel here")


# speedup vs baseline: 1.2927x; 1.2927x over previous
import jax
import jax.numpy as jnp
from jax.experimental import pallas as pl
from jax.experimental.pallas import tpu as pltpu

_N = 16384
_IN_DIM = 768
_H_DIM = 1024
_E_DIM = 256
_K = 1024
_BETA = 0.25
_BLK = 256
_GRID = _N // _BLK


def _dot(a, b, dims):
    return jax.lax.dot_general(a, b, (dims, ((), ())),
                               preferred_element_type=jnp.float32)


def _body(x_ref, W0_ref, b0_ref, W1_ref, b1_ref, W2_ref, b2_ref, E_ref,
          D0_ref, db0_ref, D1_ref, db1_ref, D2_ref, db2_ref,
          xhat_ref, idx_ref, sse_ref, counts_ref, vq_ref, ppl_ref):
    i = pl.program_id(0)

    x = x_ref[...]
    h = jnp.maximum(_dot(x, W0_ref[...], ((1,), (0,))) + b0_ref[...], 0.0)
    h = jnp.maximum(_dot(h, W1_ref[...], ((1,), (0,))) + b1_ref[...], 0.0)
    z_e = _dot(h, W2_ref[...], ((1,), (0,))) + b2_ref[...]

    E = E_ref[...]
    s1 = jnp.sum(z_e * z_e, axis=1, keepdims=True)
    s2 = jnp.sum(E * E, axis=1)[None, :]
    M = _dot(z_e, E, ((1,), (1,)))
    d = s1 + s2 - 2.0 * M

    dmin = jnp.min(d, axis=1, keepdims=True)
    iota = jax.lax.broadcasted_iota(jnp.int32, (_BLK, _K), 1)
    idx = jnp.min(jnp.where(d <= dmin, iota, _K), axis=1).astype(jnp.int32)
    idx_ref[...] = idx

    one_hot = (idx[:, None] == iota).astype(jnp.float32)
    z_q = _dot(one_hot, E, ((1,), (0,)))

    diff = z_e - z_q
    blk_sse = jnp.sum(diff * diff)
    z_q_st = z_e + (z_q - z_e)

    g = jnp.maximum(_dot(z_q_st, D0_ref[...], ((1,), (0,))) + db0_ref[...], 0.0)
    g = jnp.maximum(_dot(g, D1_ref[...], ((1,), (0,))) + db1_ref[...], 0.0)
    xhat_ref[...] = _dot(g, D2_ref[...], ((1,), (0,))) + db2_ref[...]

    @pl.when(i == 0)
    def _init():
        sse_ref[...] = jnp.zeros_like(sse_ref)
        counts_ref[...] = jnp.zeros_like(counts_ref)

    sse_ref[...] += blk_sse[None, None]
    counts_ref[...] += jnp.sum(one_hot, axis=0)[None, :]

    @pl.when(i == _GRID - 1)
    def _final():
        sse = sse_ref[0, 0]
        vq_ref[...] = ((1.0 + _BETA) * (sse / (_N * _E_DIM)))[None, None]
        p = counts_ref[...] * (1.0 / _N)
        ent = jnp.sum(p * jnp.log(p + 1e-10))
        ppl_ref[...] = jnp.exp(-ent)[None, None]


def kernel(x, W0, b0, W1, b1, W2, b2, E, D0, db0, D1, db1, D2, db2):
    b0r, b1r, b2r = b0[None, :], b1[None, :], b2[None, :]
    db0r, db1r, db2r = db0[None, :], db1[None, :], db2[None, :]

    full = lambda s: pl.BlockSpec(s, lambda i: (0, 0))
    out_shapes = (
        jax.ShapeDtypeStruct((_N, _IN_DIM), jnp.float32),
        jax.ShapeDtypeStruct((_N,), jnp.int32),
        jax.ShapeDtypeStruct((1, 1), jnp.float32),
        jax.ShapeDtypeStruct((1, _K), jnp.float32),
        jax.ShapeDtypeStruct((1, 1), jnp.float32),
        jax.ShapeDtypeStruct((1, 1), jnp.float32),
    )
    grid_spec = pl.GridSpec(
        grid=(_GRID,),
        in_specs=[
            pl.BlockSpec((_BLK, _IN_DIM), lambda i: (i, 0)),
            full((_IN_DIM, _H_DIM)), full((1, _H_DIM)),
            full((_H_DIM, _H_DIM)), full((1, _H_DIM)),
            full((_H_DIM, _E_DIM)), full((1, _E_DIM)),
            full((_K, _E_DIM)),
            full((_E_DIM, _H_DIM)), full((1, _H_DIM)),
            full((_H_DIM, _H_DIM)), full((1, _H_DIM)),
            full((_H_DIM, _IN_DIM)), full((1, _IN_DIM)),
        ],
        out_specs=[
            pl.BlockSpec((_BLK, _IN_DIM), lambda i: (i, 0)),
            pl.BlockSpec((_BLK,), lambda i: (i,)),
            full((1, 1)),
            full((1, _K)),
            full((1, 1)),
            full((1, 1)),
        ],
    )
    x_hat, indices, _sse, _counts, vq, ppl = pl.pallas_call(
        _body,
        grid_spec=grid_spec,
        out_shape=out_shapes,
        compiler_params=pltpu.CompilerParams(
            dimension_semantics=("arbitrary",),
        ),
    )(x, W0, b0r, W1, b1r, W2, b2r, E, D0, db0r, D1, db1r, D2, db2r)
    return (vq[0, 0], x_hat, ppl[0, 0], indices)


# speedup vs baseline: 1.4888x; 1.1517x over previous
import jax
import jax.numpy as jnp
from jax.experimental import pallas as pl
from jax.experimental.pallas import tpu as pltpu

_N = 16384
_IN_DIM = 768
_H_DIM = 1024
_E_DIM = 256
_K = 1024
_BETA = 0.25
_BLK = 512
_GRID = _N // _BLK


def _dot(a, b, dims):
    return jax.lax.dot_general(a, b, (dims, ((), ())),
                               preferred_element_type=jnp.float32)


def _body(x_ref, W0_ref, b0_ref, W1_ref, b1_ref, W2_ref, b2_ref, E_ref,
          D0_ref, db0_ref, D1_ref, db1_ref, D2_ref, db2_ref,
          xhat_ref, idx_ref, sse_ref, counts_ref, vq_ref, ppl_ref):
    i = pl.program_id(0)

    x = x_ref[...]
    h = jnp.maximum(_dot(x, W0_ref[...], ((1,), (0,))) + b0_ref[...], 0.0)
    h = jnp.maximum(_dot(h, W1_ref[...], ((1,), (0,))) + b1_ref[...], 0.0)
    z_e = _dot(h, W2_ref[...], ((1,), (0,))) + b2_ref[...]

    E = E_ref[...]
    s1 = jnp.sum(z_e * z_e, axis=1, keepdims=True)
    s2 = jnp.sum(E * E, axis=1)[None, :]
    M = _dot(z_e, E, ((1,), (1,)))
    d = s1 + s2 - 2.0 * M

    dmin = jnp.min(d, axis=1, keepdims=True)
    iota = jax.lax.broadcasted_iota(jnp.int32, (_BLK, _K), 1)
    idx = jnp.min(jnp.where(d <= dmin, iota, _K), axis=1).astype(jnp.int32)
    idx_ref[...] = idx

    one_hot = (idx[:, None] == iota).astype(jnp.float32)
    z_q = _dot(one_hot, E, ((1,), (0,)))

    diff = z_e - z_q
    blk_sse = jnp.sum(diff * diff)
    z_q_st = z_e + (z_q - z_e)

    g = jnp.maximum(_dot(z_q_st, D0_ref[...], ((1,), (0,))) + db0_ref[...], 0.0)
    g = jnp.maximum(_dot(g, D1_ref[...], ((1,), (0,))) + db1_ref[...], 0.0)
    xhat_ref[...] = _dot(g, D2_ref[...], ((1,), (0,))) + db2_ref[...]

    @pl.when(i == 0)
    def _init():
        sse_ref[...] = jnp.zeros_like(sse_ref)
        counts_ref[...] = jnp.zeros_like(counts_ref)

    sse_ref[...] += blk_sse[None, None]
    counts_ref[...] += jnp.sum(one_hot, axis=0)[None, :]

    @pl.when(i == _GRID - 1)
    def _final():
        sse = sse_ref[0, 0]
        vq_ref[...] = ((1.0 + _BETA) * (sse / (_N * _E_DIM)))[None, None]
        p = counts_ref[...] * (1.0 / _N)
        ent = jnp.sum(p * jnp.log(p + 1e-10))
        ppl_ref[...] = jnp.exp(-ent)[None, None]


def kernel(x, W0, b0, W1, b1, W2, b2, E, D0, db0, D1, db1, D2, db2):
    b0r, b1r, b2r = b0[None, :], b1[None, :], b2[None, :]
    db0r, db1r, db2r = db0[None, :], db1[None, :], db2[None, :]

    full = lambda s: pl.BlockSpec(s, lambda i: (0, 0))
    out_shapes = (
        jax.ShapeDtypeStruct((_N, _IN_DIM), jnp.float32),
        jax.ShapeDtypeStruct((_N,), jnp.int32),
        jax.ShapeDtypeStruct((1, 1), jnp.float32),
        jax.ShapeDtypeStruct((1, _K), jnp.float32),
        jax.ShapeDtypeStruct((1, 1), jnp.float32),
        jax.ShapeDtypeStruct((1, 1), jnp.float32),
    )
    grid_spec = pl.GridSpec(
        grid=(_GRID,),
        in_specs=[
            pl.BlockSpec((_BLK, _IN_DIM), lambda i: (i, 0)),
            full((_IN_DIM, _H_DIM)), full((1, _H_DIM)),
            full((_H_DIM, _H_DIM)), full((1, _H_DIM)),
            full((_H_DIM, _E_DIM)), full((1, _E_DIM)),
            full((_K, _E_DIM)),
            full((_E_DIM, _H_DIM)), full((1, _H_DIM)),
            full((_H_DIM, _H_DIM)), full((1, _H_DIM)),
            full((_H_DIM, _IN_DIM)), full((1, _IN_DIM)),
        ],
        out_specs=[
            pl.BlockSpec((_BLK, _IN_DIM), lambda i: (i, 0)),
            pl.BlockSpec((_BLK,), lambda i: (i,)),
            full((1, 1)),
            full((1, _K)),
            full((1, 1)),
            full((1, 1)),
        ],
    )
    x_hat, indices, _sse, _counts, vq, ppl = pl.pallas_call(
        _body,
        grid_spec=grid_spec,
        out_shape=out_shapes,
        compiler_params=pltpu.CompilerParams(
            dimension_semantics=("arbitrary",),
        ),
    )(x, W0, b0r, W1, b1r, W2, b2r, E, D0, db0r, D1, db1r, D2, db2r)
    return (vq[0, 0], x_hat, ppl[0, 0], indices)


# speedup vs baseline: 1.6038x; 1.0772x over previous
import jax
import jax.numpy as jnp
from jax.experimental import pallas as pl
from jax.experimental.pallas import tpu as pltpu

_N = 16384
_IN_DIM = 768
_H_DIM = 1024
_E_DIM = 256
_K = 1024
_BETA = 0.25
_BLK = 1024
_GRID = _N // _BLK


def _dot(a, b, dims):
    return jax.lax.dot_general(a, b, (dims, ((), ())),
                               preferred_element_type=jnp.float32)


def _body(x_ref, W0_ref, b0_ref, W1_ref, b1_ref, W2_ref, b2_ref, E_ref,
          D0_ref, db0_ref, D1_ref, db1_ref, D2_ref, db2_ref,
          xhat_ref, idx_ref, sse_ref, counts_ref, vq_ref, ppl_ref):
    i = pl.program_id(0)

    x = x_ref[...]
    h = jnp.maximum(_dot(x, W0_ref[...], ((1,), (0,))) + b0_ref[...], 0.0)
    h = jnp.maximum(_dot(h, W1_ref[...], ((1,), (0,))) + b1_ref[...], 0.0)
    z_e = _dot(h, W2_ref[...], ((1,), (0,))) + b2_ref[...]

    E = E_ref[...]
    s1 = jnp.sum(z_e * z_e, axis=1, keepdims=True)
    s2 = jnp.sum(E * E, axis=1)[None, :]
    M = _dot(z_e, E, ((1,), (1,)))
    d = s1 + s2 - 2.0 * M

    dmin = jnp.min(d, axis=1, keepdims=True)
    iota = jax.lax.broadcasted_iota(jnp.int32, (_BLK, _K), 1)
    idx = jnp.min(jnp.where(d <= dmin, iota, _K), axis=1).astype(jnp.int32)
    idx_ref[...] = idx

    one_hot = (idx[:, None] == iota).astype(jnp.float32)
    z_q = _dot(one_hot, E, ((1,), (0,)))

    diff = z_e - z_q
    blk_sse = jnp.sum(diff * diff)
    z_q_st = z_e + (z_q - z_e)

    g = jnp.maximum(_dot(z_q_st, D0_ref[...], ((1,), (0,))) + db0_ref[...], 0.0)
    g = jnp.maximum(_dot(g, D1_ref[...], ((1,), (0,))) + db1_ref[...], 0.0)
    xhat_ref[...] = _dot(g, D2_ref[...], ((1,), (0,))) + db2_ref[...]

    @pl.when(i == 0)
    def _init():
        sse_ref[...] = jnp.zeros_like(sse_ref)
        counts_ref[...] = jnp.zeros_like(counts_ref)

    sse_ref[...] += blk_sse[None, None]
    counts_ref[...] += jnp.sum(one_hot, axis=0)[None, :]

    @pl.when(i == _GRID - 1)
    def _final():
        sse = sse_ref[0, 0]
        vq_ref[...] = ((1.0 + _BETA) * (sse / (_N * _E_DIM)))[None, None]
        p = counts_ref[...] * (1.0 / _N)
        ent = jnp.sum(p * jnp.log(p + 1e-10))
        ppl_ref[...] = jnp.exp(-ent)[None, None]


def kernel(x, W0, b0, W1, b1, W2, b2, E, D0, db0, D1, db1, D2, db2):
    b0r, b1r, b2r = b0[None, :], b1[None, :], b2[None, :]
    db0r, db1r, db2r = db0[None, :], db1[None, :], db2[None, :]

    full = lambda s: pl.BlockSpec(s, lambda i: (0, 0))
    out_shapes = (
        jax.ShapeDtypeStruct((_N, _IN_DIM), jnp.float32),
        jax.ShapeDtypeStruct((_N,), jnp.int32),
        jax.ShapeDtypeStruct((1, 1), jnp.float32),
        jax.ShapeDtypeStruct((1, _K), jnp.float32),
        jax.ShapeDtypeStruct((1, 1), jnp.float32),
        jax.ShapeDtypeStruct((1, 1), jnp.float32),
    )
    grid_spec = pl.GridSpec(
        grid=(_GRID,),
        in_specs=[
            pl.BlockSpec((_BLK, _IN_DIM), lambda i: (i, 0)),
            full((_IN_DIM, _H_DIM)), full((1, _H_DIM)),
            full((_H_DIM, _H_DIM)), full((1, _H_DIM)),
            full((_H_DIM, _E_DIM)), full((1, _E_DIM)),
            full((_K, _E_DIM)),
            full((_E_DIM, _H_DIM)), full((1, _H_DIM)),
            full((_H_DIM, _H_DIM)), full((1, _H_DIM)),
            full((_H_DIM, _IN_DIM)), full((1, _IN_DIM)),
        ],
        out_specs=[
            pl.BlockSpec((_BLK, _IN_DIM), lambda i: (i, 0)),
            pl.BlockSpec((_BLK,), lambda i: (i,)),
            full((1, 1)),
            full((1, _K)),
            full((1, 1)),
            full((1, 1)),
        ],
    )
    x_hat, indices, _sse, _counts, vq, ppl = pl.pallas_call(
        _body,
        grid_spec=grid_spec,
        out_shape=out_shapes,
        compiler_params=pltpu.CompilerParams(
            dimension_semantics=("arbitrary",),
        ),
    )(x, W0, b0r, W1, b1r, W2, b2r, E, D0, db0r, D1, db1r, D2, db2r)
    return (vq[0, 0], x_hat, ppl[0, 0], indices)
